# Initial kernel scaffold; baseline (speedup 1.0000x reference)
import jax
import jax.numpy as jnp
from jax.experimental import pallas as pl


def kernel(src_xyz, src_desc, dst_xyz, dst_desc, src_weights, dst_weights, W2_0, W2_1, W2_2, g2_0, b2_0, g2_1, b2_1, g2_2, b2_2, W1_0, W1_1, W1_2, g1_0, b1_0, g1_1, b1_1, g1_2, b1_2, Wm1, bm1, gm1, bnm1, Wm2, bm2, gm2, bnm2, Wm3, bm3):
    raise NotImplementedError("write your pallas kernel here")


# speedup vs baseline: 4.3064x; 4.3064x over previous
import functools

import jax
import jax.numpy as jnp
from jax.experimental import pallas as pl
from jax.experimental.pallas import tpu as pltpu

B, N, C, K = 2, 512, 128, 16
EPS = 1e-5
P2D = B * K * N
P1D = B * N


def _dot(a, b, dims):
    return jax.lax.dot_general(a, b, (dims, ((), ())),
                               preferred_element_type=jnp.float32)


def _dot_hi(a, b, dims):
    return jax.lax.dot_general(a, b, (dims, ((), ())),
                               precision=jax.lax.Precision.HIGHEST,
                               preferred_element_type=jnp.float32)


def _topk_min(d, k):
    n = d.shape[1]
    col_i = jax.lax.broadcasted_iota(jnp.int32, d.shape, 1)
    idxs = []
    for _ in range(k):
        m = jnp.min(d, axis=1, keepdims=True)
        cand = jnp.where(d == m, col_i, n)
        ij = jnp.min(cand, axis=1, keepdims=True)
        idxs.append(ij)
        d = jnp.where(col_i == ij, jnp.float32(jnp.inf), d)
    return idxs


def _col_j(idxv, j):
    lane_i = jax.lax.broadcasted_iota(jnp.int32, idxv.shape, 1)
    return jnp.sum(jnp.where(lane_i == j, idxv, 0), axis=1, keepdims=True)


def _desc_knn_body(src_ref, dst_ref, dxyz_ref, dw_ref, sxyz_ref,
                   idx_ref, sdk_ref, ddk_ref, kdesc_ref, kxyz_ref,
                   krel_ref, kdist_ref, kw_ref, idx_s):
    j = pl.program_id(1)
    col_i = jax.lax.broadcasted_iota(jnp.int32, (N, N), 1)
    dst = dst_ref[0]

    @pl.when(j == 0)
    def _prep():
        src = src_ref[0]
        S = _dot(src, dst, ((1,), (1,)))
        ns2 = jnp.sum(src * src, axis=1, keepdims=True)
        nd2_row = _dot_hi(jnp.ones((1, C), jnp.float32), dst * dst,
                          ((1,), (1,)))
        d = ns2 + nd2_row - 2.0 * S
        idxs = _topk_min(d, K)
        idxc = jnp.concatenate(idxs, axis=1)
        idx_s[...] = idxc
        idx_ref[0] = idxc

        ns = jnp.sqrt(ns2)
        nd_row = jnp.sqrt(nd2_row)
        cm = S / (ns * nd_row + 1e-6)
        rowmax = jnp.max(cm, axis=1, keepdims=True)
        colmax = jnp.max(cm, axis=0, keepdims=True)
        gjs, cgs = [], []
        for jj in range(K):
            oh = (col_i == idxs[jj]).astype(jnp.float32)
            gjs.append(jnp.sum(cm * oh, axis=1, keepdims=True))
            cgs.append(jnp.sum(colmax * oh, axis=1, keepdims=True))
        gj = jnp.concatenate(gjs, axis=1)
        cg = jnp.concatenate(cgs, axis=1)
        sdk_ref[0] = gj / (rowmax + 1e-6)
        ddk_ref[0] = gj / (cg + 1e-6)

    ij = _col_j(idx_s[...], j)
    oh = (col_i == ij).astype(jnp.float32)
    kdesc_ref[0, 0] = _dot(oh, dst, ((1,), (0,)))
    gxyz = _dot(oh, dxyz_ref[0], ((1,), (0,)))
    kxyz_ref[0, 0] = gxyz
    rel = gxyz - sxyz_ref[0]
    krel_ref[0, 0] = rel
    kdist_ref[0, 0] = jnp.sqrt(jnp.sum(rel * rel, axis=1, keepdims=True))
    kw_ref[0, 0] = _dot(oh, dw_ref[0], ((1,), (0,)))


def _desc_knn(src_desc_t, dst_desc_t, dst_xyz, dst_w, src_xyz):
    f32 = jnp.float32
    outs = (
        jax.ShapeDtypeStruct((B, N, K), jnp.int32),
        jax.ShapeDtypeStruct((B, N, K), f32),
        jax.ShapeDtypeStruct((B, N, K), f32),
        jax.ShapeDtypeStruct((B, K, N, C), f32),
        jax.ShapeDtypeStruct((B, K, N, 3), f32),
        jax.ShapeDtypeStruct((B, K, N, 3), f32),
        jax.ShapeDtypeStruct((B, K, N, 1), f32),
        jax.ShapeDtypeStruct((B, K, N, 1), f32),
    )
    i3 = lambda b, j: (b, 0, 0)
    i4 = lambda b, j: (b, j, 0, 0)
    return pl.pallas_call(
        _desc_knn_body,
        grid=(B, K),
        in_specs=[
            pl.BlockSpec((1, N, C), i3),
            pl.BlockSpec((1, N, C), i3),
            pl.BlockSpec((1, N, 3), i3),
            pl.BlockSpec((1, N, 1), i3),
            pl.BlockSpec((1, N, 3), i3),
        ],
        out_specs=[
            pl.BlockSpec((1, N, K), i3),
            pl.BlockSpec((1, N, K), i3),
            pl.BlockSpec((1, N, K), i3),
            pl.BlockSpec((1, 1, N, C), i4),
            pl.BlockSpec((1, 1, N, 3), i4),
            pl.BlockSpec((1, 1, N, 3), i4),
            pl.BlockSpec((1, 1, N, 1), i4),
            pl.BlockSpec((1, 1, N, 1), i4),
        ],
        out_shape=outs,
        scratch_shapes=[pltpu.VMEM((N, K), jnp.int32)],
    )(src_desc_t, dst_desc_t, dst_xyz, dst_w, src_xyz)


def _xyz_knn_body(xyz_ref, desc_ref, gdesc_ref, rel_ref, dist_ref, idx_s):
    j = pl.program_id(1)
    xyz = xyz_ref[0]
    col_i = jax.lax.broadcasted_iota(jnp.int32, (N, N), 1)

    @pl.when(j == 0)
    def _prep():
        Pm = _dot(xyz, xyz, ((1,), (1,)))
        ss = jnp.sum(xyz * xyz, axis=1, keepdims=True)
        ss_row = _dot_hi(jnp.ones((1, 3), jnp.float32), xyz * xyz,
                         ((1,), (1,)))
        d = ss + ss_row - 2.0 * Pm
        idx_s[...] = jnp.concatenate(_topk_min(d, K), axis=1)

    ij = _col_j(idx_s[...], j)
    oh = (col_i == ij).astype(jnp.float32)
    gdesc_ref[0, 0] = _dot(oh, desc_ref[0], ((1,), (0,)))
    gxyz = _dot(oh, xyz, ((1,), (0,)))
    rel = gxyz - xyz
    rel_ref[0, 0] = rel
    dist_ref[0, 0] = jnp.sqrt(jnp.sum(rel * rel, axis=1, keepdims=True))


def _xyz_knn(xyz, desc_t):
    f32 = jnp.float32
    outs = (
        jax.ShapeDtypeStruct((B, K, N, C), f32),
        jax.ShapeDtypeStruct((B, K, N, 3), f32),
        jax.ShapeDtypeStruct((B, K, N, 1), f32),
    )
    i3 = lambda b, j: (b, 0, 0)
    i4 = lambda b, j: (b, j, 0, 0)
    return pl.pallas_call(
        _xyz_knn_body,
        grid=(B, K),
        in_specs=[pl.BlockSpec((1, N, 3), i3), pl.BlockSpec((1, N, C), i3)],
        out_specs=[
            pl.BlockSpec((1, 1, N, C), i4),
            pl.BlockSpec((1, 1, N, 3), i4),
            pl.BlockSpec((1, 1, N, 1), i4),
        ],
        out_shape=outs,
        scratch_shapes=[pltpu.VMEM((N, K), jnp.int32)],
    )(xyz, desc_t)


def _nbr_cos_body(s_ref, d_ref, idx_ref, sdnk_ref, dsnk_ref):
    s = s_ref[0]
    t = d_ref[0]
    S2 = _dot(s, t, ((1,), (1,)))
    ns = jnp.sqrt(jnp.sum(s * s, axis=1, keepdims=True))
    nd_row = jnp.sqrt(_dot_hi(jnp.ones((1, C), jnp.float32), t * t,
                              ((1,), (1,))))
    cm = S2 / (ns * nd_row + 1e-6)
    rowmax = jnp.max(cm, axis=1, keepdims=True)
    colmax = jnp.max(cm, axis=0, keepdims=True)
    idx = idx_ref[0]
    col_i = jax.lax.broadcasted_iota(jnp.int32, (N, N), 1)
    gjs, cgs = [], []
    for j in range(K):
        ij = idx[:, j:j + 1]
        oh = (col_i == ij).astype(jnp.float32)
        gjs.append(jnp.sum(cm * oh, axis=1, keepdims=True))
        cgs.append(jnp.sum(colmax * oh, axis=1, keepdims=True))
    gj = jnp.concatenate(gjs, axis=1)
    cg = jnp.concatenate(cgs, axis=1)
    sdnk_ref[0] = gj / (rowmax + 1e-6)
    dsnk_ref[0] = gj / (cg + 1e-6)


def _nbr_cos(src_nbr_desc, dst_nbr_desc, idx):
    f32 = jnp.float32
    i3 = lambda b: (b, 0, 0)
    i4 = lambda b: (b, 0, 0, 0)
    return pl.pallas_call(
        _nbr_cos_body,
        grid=(B,),
        in_specs=[
            pl.BlockSpec((1, N, C), i3),
            pl.BlockSpec((1, N, C), i3),
            pl.BlockSpec((1, N, K), i3),
        ],
        out_specs=[
            pl.BlockSpec((1, N, K), i3),
            pl.BlockSpec((1, N, K), i3),
        ],
        out_shape=(jax.ShapeDtypeStruct((B, N, K), f32),
                   jax.ShapeDtypeStruct((B, N, K), f32)),
    )(src_nbr_desc, dst_nbr_desc, idx)


def _bn_stats_from(stats, n_pos):
    st = jnp.sum(stats, axis=0)
    mean = st[0:1, :] / n_pos
    var = st[1:2, :] / n_pos - mean * mean
    inv = 1.0 / jnp.sqrt(var + EPS)
    return mean, inv


def _layer_body(prenorm, has_bias, n_pos, *refs):
    if prenorm:
        if has_bias:
            (x_ref, w_ref, bias_ref, st_in_ref, g_ref, b_ref,
             y_ref, st_ref) = refs
        else:
            x_ref, w_ref, st_in_ref, g_ref, b_ref, y_ref, st_ref = refs
    else:
        if has_bias:
            x_ref, w_ref, bias_ref, y_ref, st_ref = refs
        else:
            x_ref, w_ref, y_ref, st_ref = refs
    x = x_ref[...]
    if prenorm:
        mean, inv = _bn_stats_from(st_in_ref[...], n_pos)
        x = jnp.maximum((x - mean) * inv * g_ref[...] + b_ref[...], 0.0)
    y = _dot(x, w_ref[...], ((1,), (1,)))
    if has_bias:
        y = y + bias_ref[...]
    y_ref[...] = y
    s0 = jnp.sum(y, axis=0, keepdims=True)
    s1 = jnp.sum(y * y, axis=0, keepdims=True)
    st_ref[0] = jnp.concatenate([s0, s1], axis=0)


def _layer(x, w, bias=None, stats_in=None, g=None, b=None, tile=2048):
    n_pos, cin = x.shape
    cout = w.shape[0]
    nt = n_pos // tile
    prenorm = stats_in is not None
    has_bias = bias is not None
    f32 = jnp.float32
    in_specs = [pl.BlockSpec((tile, cin), lambda i: (i, 0)),
                pl.BlockSpec((cout, cin), lambda i: (0, 0))]
    args = [x, w]
    if has_bias:
        in_specs.append(pl.BlockSpec((1, cout), lambda i: (0, 0)))
        args.append(bias.reshape(1, cout))
    if prenorm:
        nt_in = stats_in.shape[0]
        in_specs += [pl.BlockSpec((nt_in, 2, cin), lambda i: (0, 0, 0)),
                     pl.BlockSpec((1, cin), lambda i: (0, 0)),
                     pl.BlockSpec((1, cin), lambda i: (0, 0))]
        args += [stats_in, g.reshape(1, cin), b.reshape(1, cin)]
    return pl.pallas_call(
        functools.partial(_layer_body, prenorm, has_bias, float(n_pos)),
        grid=(nt,),
        in_specs=in_specs,
        out_specs=[pl.BlockSpec((tile, cout), lambda i: (i, 0)),
                   pl.BlockSpec((1, 2, cout), lambda i: (i, 0, 0))],
        out_shape=(jax.ShapeDtypeStruct((n_pos, cout), f32),
                   jax.ShapeDtypeStruct((nt, 2, cout), f32)),
    )(*args)


def _attn_weights(y, mean, inv, g, b, cdim):
    def xn_j(j):
        blk = y[j * N:(j + 1) * N, :]
        return jnp.maximum((blk - mean) * inv * g + b, 0.0)
    cms = []
    for j in range(K):
        cms.append(jnp.max(xn_j(j), axis=1, keepdims=True))
    a = jnp.concatenate(cms, axis=1)
    m = jnp.max(a, axis=1, keepdims=True)
    e = jnp.exp(a - m)
    aw = e / jnp.sum(e, axis=1, keepdims=True)
    return aw, xn_j


def _attn2_body(n_pos, y_ref, st_ref, g_ref, b_ref, feats_ref, out_ref):
    mean, inv = _bn_stats_from(st_ref[...], n_pos)
    aw, _ = _attn_weights(y_ref[0], mean, inv, g_ref[...], b_ref[...], C)
    acc = jnp.zeros((N, C), jnp.float32)
    for j in range(K):
        acc = acc + aw[:, j:j + 1] * feats_ref[0, j]
    out_ref[0] = acc


def _attn2(y, stats, g, b, gdesc):
    f32 = jnp.float32
    y3 = y.reshape(B, K * N, C)
    return pl.pallas_call(
        functools.partial(_attn2_body, float(P2D)),
        grid=(B,),
        in_specs=[
            pl.BlockSpec((1, K * N, C), lambda bb: (bb, 0, 0)),
            pl.BlockSpec(stats.shape, lambda bb: (0, 0, 0)),
            pl.BlockSpec((1, C), lambda bb: (0, 0)),
            pl.BlockSpec((1, C), lambda bb: (0, 0)),
            pl.BlockSpec((1, K, N, C), lambda bb: (bb, 0, 0, 0)),
        ],
        out_specs=pl.BlockSpec((1, N, C), lambda bb: (bb, 0, 0)),
        out_shape=jax.ShapeDtypeStruct((B, N, C), f32),
    )(y3, stats, g.reshape(1, C), b.reshape(1, C), gdesc)


def _attn1_body(n_pos, cdim, y_ref, st_ref, g_ref, b_ref, kxyz_ref,
                corres_ref, att_ref):
    mean, inv = _bn_stats_from(st_ref[...], n_pos)
    aw, xn_j = _attn_weights(y_ref[0], mean, inv, g_ref[...], b_ref[...],
                             cdim)
    corres = jnp.zeros((N, 3), jnp.float32)
    att = jnp.zeros((N, cdim), jnp.float32)
    for j in range(K):
        awj = aw[:, j:j + 1]
        corres = corres + awj * kxyz_ref[0, j]
        att = att + awj * xn_j(j)
    corres_ref[0] = corres
    att_ref[0] = att


def _attn1(y, stats, g, b, kxyz, cdim):
    f32 = jnp.float32
    y3 = y.reshape(B, K * N, cdim)
    return pl.pallas_call(
        functools.partial(_attn1_body, float(P2D), cdim),
        grid=(B,),
        in_specs=[
            pl.BlockSpec((1, K * N, cdim), lambda bb: (bb, 0, 0)),
            pl.BlockSpec(stats.shape, lambda bb: (0, 0, 0)),
            pl.BlockSpec((1, cdim), lambda bb: (0, 0)),
            pl.BlockSpec((1, cdim), lambda bb: (0, 0)),
            pl.BlockSpec((1, K, N, 3), lambda bb: (bb, 0, 0, 0)),
        ],
        out_specs=[
            pl.BlockSpec((1, N, 3), lambda bb: (bb, 0, 0)),
            pl.BlockSpec((1, N, cdim), lambda bb: (bb, 0, 0)),
        ],
        out_shape=(jax.ShapeDtypeStruct((B, N, 3), f32),
                   jax.ShapeDtypeStruct((B, N, cdim), f32)),
    )(y3, stats, g.reshape(1, cdim), b.reshape(1, cdim), kxyz)


def _head_body(x_ref, w1_ref, b1_ref, g1_ref, n1_ref, w2_ref, b2_ref,
               g2_ref, n2_ref, w3_ref, b3_ref, out_ref):
    def cbr(x, w_ref, bias_ref, g_ref, bt_ref):
        y = _dot(x, w_ref[...], ((1,), (1,))) + bias_ref[...]
        mean = jnp.mean(y, axis=0, keepdims=True)
        var = jnp.mean((y - mean) * (y - mean), axis=0, keepdims=True)
        xn = (y - mean) / jnp.sqrt(var + EPS) * g_ref[...] + bt_ref[...]
        return jnp.maximum(xn, 0.0)

    x = cbr(x_ref[...], w1_ref, b1_ref, g1_ref, n1_ref)
    x = cbr(x, w2_ref, b2_ref, g2_ref, n2_ref)
    w = jnp.sum(x * w3_ref[...], axis=1, keepdims=True) + b3_ref[0, 0]
    out_ref[...] = 1.0 / (1.0 + jnp.exp(-w))


def _head(att, wm1, bm1, gm1, bnm1, wm2, bm2, gm2, bnm2, wm3, bm3):
    c2 = att.shape[1]
    f32 = jnp.float32
    r = lambda v: v.reshape(1, -1)
    return pl.pallas_call(
        _head_body,
        out_shape=jax.ShapeDtypeStruct((P1D, 1), f32),
    )(att, wm1, r(bm1), r(gm1), r(bnm1), wm2, r(bm2), r(gm2), r(bnm2),
      wm3, r(bm3))


def kernel(src_xyz, src_desc, dst_xyz, dst_desc, src_weights, dst_weights,
           W2_0, W2_1, W2_2, g2_0, b2_0, g2_1, b2_1, g2_2, b2_2,
           W1_0, W1_1, W1_2, g1_0, b1_0, g1_1, b1_1, g1_2, b1_2,
           Wm1, bm1, gm1, bnm1, Wm2, bm2, gm2, bnm2, Wm3, bm3):
    src_desc_t = jnp.transpose(src_desc, (0, 2, 1))
    dst_desc_t = jnp.transpose(dst_desc, (0, 2, 1))

    (idx, sdk, ddk, kdesc, kxyz, krel, kdist, kw) = _desc_knn(
        src_desc_t, dst_desc_t, dst_xyz, dst_weights[..., None], src_xyz)

    s_gdesc, s_rel, s_dist = _xyz_knn(src_xyz, src_desc_t)
    d_gdesc, d_rel, d_dist = _xyz_knn(dst_xyz, dst_desc_t)

    def convs2(gdesc, rel, dist):
        feats = jnp.concatenate([gdesc, rel, dist], axis=-1)
        x0 = feats.reshape(P2D, C + 4)
        y0, st0 = _layer(x0, W2_0)
        y1, st1 = _layer(y0, W2_1, stats_in=st0, g=g2_0, b=b2_0)
        y2, st2 = _layer(y1, W2_2, stats_in=st1, g=g2_1, b=b2_1)
        return _attn2(y2, st2, g2_2, b2_2, gdesc)

    src_nbr_desc = convs2(s_gdesc, s_rel, s_dist)
    dst_nbr_desc = convs2(d_gdesc, d_rel, d_dist)

    sdnk, dsnk = _nbr_cos(src_nbr_desc, dst_nbr_desc, idx)

    bc = lambda v, ch: jnp.broadcast_to(v[:, None], (B, K, N, ch))
    tr = lambda v: jnp.transpose(v, (0, 2, 1))[..., None]
    feats2 = jnp.concatenate([
        krel, kdist, bc(src_xyz, 3), kxyz,
        bc(src_desc_t, C), kdesc, bc(src_weights[..., None], 1), kw,
        tr(sdk), tr(ddk), tr(sdnk), tr(dsnk),
    ], axis=-1)
    c2 = 2 * C
    x0 = feats2.reshape(P2D, c2 + 16)
    y0, st0 = _layer(x0, W1_0)
    y1, st1 = _layer(y0, W1_1, stats_in=st0, g=g1_0, b=b1_0)
    y2, st2 = _layer(y1, W1_2, stats_in=st1, g=g1_1, b=b1_1)
    corres_xyz, att = _attn1(y2, st2, g1_2, b1_2, kxyz, c2)

    w = _head(att.reshape(P1D, c2), Wm1, bm1, gm1, bnm1,
              Wm2, bm2, gm2, bnm2, Wm3, bm3)
    weights = w.reshape(B, N)
    return corres_xyz, weights


# speedup vs baseline: 5.0363x; 1.1695x over previous
import functools

import jax
import jax.numpy as jnp
from jax.experimental import pallas as pl
from jax.experimental.pallas import tpu as pltpu

B, N, C, K = 2, 512, 128, 16
EPS = 1e-5
P2D = B * K * N
P1D = B * N


def _dot(a, b, dims):
    return jax.lax.dot_general(a, b, (dims, ((), ())),
                               preferred_element_type=jnp.float32)


def _dot_hi(a, b, dims):
    return jax.lax.dot_general(a, b, (dims, ((), ())),
                               precision=jax.lax.Precision.HIGHEST,
                               preferred_element_type=jnp.float32)


def _topk_min(d, k):
    n = d.shape[1]
    col_i = jax.lax.broadcasted_iota(jnp.int32, d.shape, 1)
    idxs = []
    for _ in range(k):
        m = jnp.min(d, axis=1, keepdims=True)
        cand = jnp.where(d == m, col_i, n)
        ij = jnp.min(cand, axis=1, keepdims=True)
        idxs.append(ij)
        d = jnp.where(col_i == ij, jnp.float32(jnp.inf), d)
    return idxs


def _col_j(idxv, j):
    lane_i = jax.lax.broadcasted_iota(jnp.int32, idxv.shape, 1)
    return jnp.sum(jnp.where(lane_i == j, idxv, 0), axis=1, keepdims=True)


def _desc_knn_body(src_ref, dst_ref, dxyz_ref, dw_ref, sxyz_ref,
                   idx_ref, sdk_ref, ddk_ref, kdesc_ref, kxyz_ref,
                   krel_ref, kdist_ref, kw_ref, idx_s):
    j = pl.program_id(1)
    col_i = jax.lax.broadcasted_iota(jnp.int32, (N, N), 1)
    dst = dst_ref[0]

    @pl.when(j == 0)
    def _prep():
        src = src_ref[0]
        S = _dot(src, dst, ((1,), (1,)))
        ns2 = jnp.sum(src * src, axis=1, keepdims=True)
        nd2_row = _dot_hi(jnp.ones((1, C), jnp.float32), dst * dst,
                          ((1,), (1,)))
        d = ns2 + nd2_row - 2.0 * S
        idxs = _topk_min(d, K)
        idxc = jnp.concatenate(idxs, axis=1)
        idx_s[...] = idxc
        idx_ref[0] = idxc

        ns = jnp.sqrt(ns2)
        nd_row = jnp.sqrt(nd2_row)
        cm = S / (ns * nd_row + 1e-6)
        rowmax = jnp.max(cm, axis=1, keepdims=True)
        colmax = jnp.max(cm, axis=0, keepdims=True)
        gjs, cgs = [], []
        for jj in range(K):
            oh = (col_i == idxs[jj]).astype(jnp.float32)
            gjs.append(jnp.sum(cm * oh, axis=1, keepdims=True))
            cgs.append(jnp.sum(colmax * oh, axis=1, keepdims=True))
        gj = jnp.concatenate(gjs, axis=1)
        cg = jnp.concatenate(cgs, axis=1)
        sdk_ref[0] = gj / (rowmax + 1e-6)
        ddk_ref[0] = gj / (cg + 1e-6)

    ij = _col_j(idx_s[...], j)
    oh = (col_i == ij).astype(jnp.float32)
    kdesc_ref[0, 0] = _dot(oh, dst, ((1,), (0,)))
    gxyz = _dot(oh, dxyz_ref[0], ((1,), (0,)))
    kxyz_ref[0, 0] = gxyz
    rel = gxyz - sxyz_ref[0]
    krel_ref[0, 0] = rel
    kdist_ref[0, 0] = jnp.sqrt(jnp.sum(rel * rel, axis=1, keepdims=True))
    kw_ref[0, 0] = _dot(oh, dw_ref[0], ((1,), (0,)))


def _desc_knn(src_desc_t, dst_desc_t, dst_xyz, dst_w, src_xyz):
    f32 = jnp.float32
    outs = (
        jax.ShapeDtypeStruct((B, N, K), jnp.int32),
        jax.ShapeDtypeStruct((B, N, K), f32),
        jax.ShapeDtypeStruct((B, N, K), f32),
        jax.ShapeDtypeStruct((B, K, N, C), f32),
        jax.ShapeDtypeStruct((B, K, N, 3), f32),
        jax.ShapeDtypeStruct((B, K, N, 3), f32),
        jax.ShapeDtypeStruct((B, K, N, 1), f32),
        jax.ShapeDtypeStruct((B, K, N, 1), f32),
    )
    i3 = lambda b, j: (b, 0, 0)
    i4 = lambda b, j: (b, j, 0, 0)
    return pl.pallas_call(
        _desc_knn_body,
        grid=(B, K),
        in_specs=[
            pl.BlockSpec((1, N, C), i3),
            pl.BlockSpec((1, N, C), i3),
            pl.BlockSpec((1, N, 3), i3),
            pl.BlockSpec((1, N, 1), i3),
            pl.BlockSpec((1, N, 3), i3),
        ],
        out_specs=[
            pl.BlockSpec((1, N, K), i3),
            pl.BlockSpec((1, N, K), i3),
            pl.BlockSpec((1, N, K), i3),
            pl.BlockSpec((1, 1, N, C), i4),
            pl.BlockSpec((1, 1, N, 3), i4),
            pl.BlockSpec((1, 1, N, 3), i4),
            pl.BlockSpec((1, 1, N, 1), i4),
            pl.BlockSpec((1, 1, N, 1), i4),
        ],
        out_shape=outs,
        scratch_shapes=[pltpu.VMEM((N, K), jnp.int32)],
    )(src_desc_t, dst_desc_t, dst_xyz, dst_w, src_xyz)


def _xyz_knn_body(xyz_ref, desc_ref, gdesc_ref, rel_ref, dist_ref, idx_s):
    j = pl.program_id(1)
    xyz = xyz_ref[0]
    col_i = jax.lax.broadcasted_iota(jnp.int32, (N, N), 1)

    @pl.when(j == 0)
    def _prep():
        Pm = _dot(xyz, xyz, ((1,), (1,)))
        ss = jnp.sum(xyz * xyz, axis=1, keepdims=True)
        ss_row = _dot_hi(jnp.ones((1, 3), jnp.float32), xyz * xyz,
                         ((1,), (1,)))
        d = ss + ss_row - 2.0 * Pm
        idx_s[...] = jnp.concatenate(_topk_min(d, K), axis=1)

    ij = _col_j(idx_s[...], j)
    oh = (col_i == ij).astype(jnp.float32)
    gdesc_ref[0, 0] = _dot(oh, desc_ref[0], ((1,), (0,)))
    gxyz = _dot(oh, xyz, ((1,), (0,)))
    rel = gxyz - xyz
    rel_ref[0, 0] = rel
    dist_ref[0, 0] = jnp.sqrt(jnp.sum(rel * rel, axis=1, keepdims=True))


def _xyz_knn(xyz, desc_t):
    g = xyz.shape[0]
    f32 = jnp.float32
    outs = (
        jax.ShapeDtypeStruct((g, K, N, C), f32),
        jax.ShapeDtypeStruct((g, K, N, 3), f32),
        jax.ShapeDtypeStruct((g, K, N, 1), f32),
    )
    i3 = lambda b, j: (b, 0, 0)
    i4 = lambda b, j: (b, j, 0, 0)
    return pl.pallas_call(
        _xyz_knn_body,
        grid=(g, K),
        in_specs=[pl.BlockSpec((1, N, 3), i3), pl.BlockSpec((1, N, C), i3)],
        out_specs=[
            pl.BlockSpec((1, 1, N, C), i4),
            pl.BlockSpec((1, 1, N, 3), i4),
            pl.BlockSpec((1, 1, N, 1), i4),
        ],
        out_shape=outs,
        scratch_shapes=[pltpu.VMEM((N, K), jnp.int32)],
    )(xyz, desc_t)


def _nbr_cos_body(s_ref, d_ref, idx_ref, sdnk_ref, dsnk_ref):
    s = s_ref[0]
    t = d_ref[0]
    S2 = _dot(s, t, ((1,), (1,)))
    ns = jnp.sqrt(jnp.sum(s * s, axis=1, keepdims=True))
    nd_row = jnp.sqrt(_dot_hi(jnp.ones((1, C), jnp.float32), t * t,
                              ((1,), (1,))))
    cm = S2 / (ns * nd_row + 1e-6)
    rowmax = jnp.max(cm, axis=1, keepdims=True)
    colmax = jnp.max(cm, axis=0, keepdims=True)
    idx = idx_ref[0]
    col_i = jax.lax.broadcasted_iota(jnp.int32, (N, N), 1)
    gjs, cgs = [], []
    for j in range(K):
        ij = idx[:, j:j + 1]
        oh = (col_i == ij).astype(jnp.float32)
        gjs.append(jnp.sum(cm * oh, axis=1, keepdims=True))
        cgs.append(jnp.sum(colmax * oh, axis=1, keepdims=True))
    gj = jnp.concatenate(gjs, axis=1)
    cg = jnp.concatenate(cgs, axis=1)
    sdnk_ref[0] = gj / (rowmax + 1e-6)
    dsnk_ref[0] = gj / (cg + 1e-6)


def _nbr_cos(src_nbr_desc, dst_nbr_desc, idx):
    f32 = jnp.float32
    i3 = lambda b: (b, 0, 0)
    i4 = lambda b: (b, 0, 0, 0)
    return pl.pallas_call(
        _nbr_cos_body,
        grid=(B,),
        in_specs=[
            pl.BlockSpec((1, N, C), i3),
            pl.BlockSpec((1, N, C), i3),
            pl.BlockSpec((1, N, K), i3),
        ],
        out_specs=[
            pl.BlockSpec((1, N, K), i3),
            pl.BlockSpec((1, N, K), i3),
        ],
        out_shape=(jax.ShapeDtypeStruct((B, N, K), f32),
                   jax.ShapeDtypeStruct((B, N, K), f32)),
    )(src_nbr_desc, dst_nbr_desc, idx)


def _bn_apply(y, g, b, n_pos):
    s0 = jnp.sum(y, axis=0, keepdims=True)
    s1 = jnp.sum(y * y, axis=0, keepdims=True)
    mean = s0 / n_pos
    var = s1 / n_pos - mean * mean
    inv = 1.0 / jnp.sqrt(var + EPS)
    return jnp.maximum((y - mean) * inv * g + b, 0.0)


def _convs2_body(x_ref, w0_ref, w1_ref, w2_ref, g0_ref, b0_ref, g1_ref,
                 b1_ref, g2_ref, b2_ref, out_ref):
    n_pos = float(P2D)
    x = x_ref[0]
    y = _dot(x, w0_ref[...], ((1,), (1,)))
    y = _bn_apply(y, g0_ref[...], b0_ref[...], n_pos)
    y = _dot(y, w1_ref[...], ((1,), (1,)))
    y = _bn_apply(y, g1_ref[...], b1_ref[...], n_pos)
    y = _dot(y, w2_ref[...], ((1,), (1,)))
    y = _bn_apply(y, g2_ref[...], b2_ref[...], n_pos)
    for b in range(B):
        cms = []
        for j in range(K):
            r0 = (b * K + j) * N
            cms.append(jnp.max(y[r0:r0 + N, :], axis=1, keepdims=True))
        a = jnp.concatenate(cms, axis=1)
        m = jnp.max(a, axis=1, keepdims=True)
        e = jnp.exp(a - m)
        aw = e / jnp.sum(e, axis=1, keepdims=True)
        acc = jnp.zeros((N, C), jnp.float32)
        for j in range(K):
            r0 = (b * K + j) * N
            acc = acc + aw[:, j:j + 1] * x[r0:r0 + N, 0:C]
        out_ref[0, b] = acc


def _convs2(feats, w0, w1, w2, g0, b0, g1, b1, g2, b2):
    f32 = jnp.float32
    r = lambda v: v.reshape(1, -1)
    cw = lambda shp: pl.BlockSpec(shp, lambda s: (0, 0))
    return pl.pallas_call(
        _convs2_body,
        grid=(2,),
        in_specs=[pl.BlockSpec((1, P2D, C + 4), lambda s: (s, 0, 0)),
                  cw(w0.shape), cw(w1.shape), cw(w2.shape),
                  cw((1, C)), cw((1, C)), cw((1, C)), cw((1, C)),
                  cw((1, C)), cw((1, C))],
        out_specs=pl.BlockSpec((1, B, N, C), lambda s: (s, 0, 0, 0)),
        out_shape=jax.ShapeDtypeStruct((2, B, N, C), f32),
    )(feats, w0, w1, w2, r(g0), r(b0), r(g1), r(b1), r(g2), r(b2))


_TILE = 2048
_NT = P2D // _TILE


def _big3_body(x_ref, w0_ref, w1_ref, w2_ref, g0_ref, b0_ref, g1_ref,
               b1_ref, y2_ref, st2_ref, ping, pong, st0, st1):
    p = pl.program_id(0)
    t = pl.program_id(1)
    n_pos = float(P2D)

    @pl.when((p == 0) & (t == 0))
    def _init():
        st0[...] = jnp.zeros_like(st0)
        st1[...] = jnp.zeros_like(st1)

    row = t * _TILE

    @pl.when(p == 0)
    def _l0():
        y = _dot(x_ref[...], w0_ref[...], ((1,), (1,)))
        ping[pl.ds(row, _TILE), :] = y
        st0[...] = st0[...] + jnp.concatenate(
            [jnp.sum(y, axis=0, keepdims=True),
             jnp.sum(y * y, axis=0, keepdims=True)], axis=0)

    @pl.when(p == 1)
    def _l1():
        st = st0[...]
        mean = st[0:1, :] / n_pos
        var = st[1:2, :] / n_pos - mean * mean
        inv = 1.0 / jnp.sqrt(var + EPS)
        xn = jnp.maximum((ping[pl.ds(row, _TILE), :] - mean) * inv
                         * g0_ref[...] + b0_ref[...], 0.0)
        y = _dot(xn, w1_ref[...], ((1,), (1,)))
        pong[pl.ds(row, _TILE), :] = y
        st1[...] = st1[...] + jnp.concatenate(
            [jnp.sum(y, axis=0, keepdims=True),
             jnp.sum(y * y, axis=0, keepdims=True)], axis=0)

    @pl.when(p == 2)
    def _l2():
        st = st1[...]
        mean = st[0:1, :] / n_pos
        var = st[1:2, :] / n_pos - mean * mean
        inv = 1.0 / jnp.sqrt(var + EPS)
        xn = jnp.maximum((pong[pl.ds(row, _TILE), :] - mean) * inv
                         * g1_ref[...] + b1_ref[...], 0.0)
        y = _dot(xn, w2_ref[...], ((1,), (1,)))
        y2_ref[...] = y
        st2_ref[0] = jnp.concatenate(
            [jnp.sum(y, axis=0, keepdims=True),
             jnp.sum(y * y, axis=0, keepdims=True)], axis=0)


def _big3(x, w0, w1, w2, g0, b0, g1, b1):
    cin = x.shape[1]
    c2 = w2.shape[0]
    f32 = jnp.float32
    r = lambda v: v.reshape(1, -1)
    cw = lambda shp: pl.BlockSpec(shp, lambda p, t: (0, 0))
    sel = lambda pp, tt: (jnp.where(pp == 0, tt, 0), 0)
    sely = lambda pp, tt: (jnp.where(pp == 2, tt, 0), 0)
    selst = lambda pp, tt: (jnp.where(pp == 2, tt, 0), 0, 0)
    return pl.pallas_call(
        _big3_body,
        grid=(3, _NT),
        in_specs=[pl.BlockSpec((_TILE, cin), sel),
                  cw(w0.shape), cw(w1.shape), cw(w2.shape),
                  cw((1, c2)), cw((1, c2)), cw((1, c2)), cw((1, c2))],
        out_specs=[pl.BlockSpec((_TILE, c2), sely),
                   pl.BlockSpec((1, 2, c2), selst)],
        out_shape=(jax.ShapeDtypeStruct((P2D, c2), f32),
                   jax.ShapeDtypeStruct((_NT, 2, c2), f32)),
        scratch_shapes=[pltpu.VMEM((P2D, c2), f32),
                        pltpu.VMEM((P2D, c2), f32),
                        pltpu.VMEM((2, c2), f32),
                        pltpu.VMEM((2, c2), f32)],
    )(x, w0, w1, w2, r(g0), r(b0), r(g1), r(b1))


def _attn1_head_body(y_ref, st_ref, g_ref, b_ref, kxyz_ref,
                     w1_ref, b1_ref, g1_ref, n1_ref,
                     w2_ref, b2_ref, g2_ref, n2_ref, w3_ref, b3_ref,
                     corres_ref, wout_ref):
    c2 = 2 * C
    n_pos = float(P2D)
    st = jnp.sum(st_ref[...], axis=0)
    mean = st[0:1, :] / n_pos
    var = st[1:2, :] / n_pos - mean * mean
    inv = 1.0 / jnp.sqrt(var + EPS)
    g = g_ref[...]
    bb = b_ref[...]
    y = y_ref[...]

    def xn_at(b, j):
        r0 = (b * K + j) * N
        return jnp.maximum((y[r0:r0 + N, :] - mean) * inv * g + bb, 0.0)

    att_rows = []
    for b in range(B):
        cms = []
        for j in range(K):
            cms.append(jnp.max(xn_at(b, j), axis=1, keepdims=True))
        a = jnp.concatenate(cms, axis=1)
        m = jnp.max(a, axis=1, keepdims=True)
        e = jnp.exp(a - m)
        aw = e / jnp.sum(e, axis=1, keepdims=True)
        corres = jnp.zeros((N, 3), jnp.float32)
        att = jnp.zeros((N, c2), jnp.float32)
        for j in range(K):
            awj = aw[:, j:j + 1]
            corres = corres + awj * kxyz_ref[0, b * K + j]
            att = att + awj * xn_at(b, j)
        corres_ref[b] = corres
        att_rows.append(att)
    x = jnp.concatenate(att_rows, axis=0)

    def cbr(xx, w_ref, bias_ref, gg_ref, bt_ref):
        yy = _dot(xx, w_ref[...], ((1,), (1,))) + bias_ref[...]
        mm = jnp.mean(yy, axis=0, keepdims=True)
        vv = jnp.mean((yy - mm) * (yy - mm), axis=0, keepdims=True)
        return jnp.maximum((yy - mm) / jnp.sqrt(vv + EPS) * gg_ref[...]
                           + bt_ref[...], 0.0)

    x = cbr(x, w1_ref, b1_ref, g1_ref, n1_ref)
    x = cbr(x, w2_ref, b2_ref, g2_ref, n2_ref)
    w = jnp.sum(x * w3_ref[...], axis=1, keepdims=True) + b3_ref[0, 0]
    s = 1.0 / (1.0 + jnp.exp(-w))
    wout_ref[...] = jnp.concatenate(
        [s[b * N:(b + 1) * N, :] for b in range(B)], axis=1)


def _attn1_head(y2, st2, g, b, kxyz, wm1, bm1, gm1, bnm1, wm2, bm2, gm2,
                bnm2, wm3, bm3):
    c2 = 2 * C
    f32 = jnp.float32
    r = lambda v: v.reshape(1, -1)
    kx = kxyz.reshape(1, B * K, N, 3)
    return pl.pallas_call(
        _attn1_head_body,
        out_shape=(jax.ShapeDtypeStruct((B, N, 3), f32),
                   jax.ShapeDtypeStruct((N, B), f32)),
    )(y2, st2, r(g), r(b), kx, wm1, r(bm1), r(gm1), r(bnm1),
      wm2, r(bm2), r(gm2), r(bnm2), wm3, r(bm3))


def kernel(src_xyz, src_desc, dst_xyz, dst_desc, src_weights, dst_weights,
           W2_0, W2_1, W2_2, g2_0, b2_0, g2_1, b2_1, g2_2, b2_2,
           W1_0, W1_1, W1_2, g1_0, b1_0, g1_1, b1_1, g1_2, b1_2,
           Wm1, bm1, gm1, bnm1, Wm2, bm2, gm2, bnm2, Wm3, bm3):
    src_desc_t = jnp.transpose(src_desc, (0, 2, 1))
    dst_desc_t = jnp.transpose(dst_desc, (0, 2, 1))

    (idx, sdk, ddk, kdesc, kxyz, krel, kdist, kw) = _desc_knn(
        src_desc_t, dst_desc_t, dst_xyz, dst_weights[..., None], src_xyz)

    xyz_st = jnp.concatenate([src_xyz, dst_xyz], axis=0)
    desc_st = jnp.concatenate([src_desc_t, dst_desc_t], axis=0)
    gdesc, grel, gdist = _xyz_knn(xyz_st, desc_st)

    feats1 = jnp.concatenate([gdesc, grel, gdist], axis=-1)
    nbr_desc = _convs2(feats1.reshape(2, P2D, C + 4),
                       W2_0, W2_1, W2_2, g2_0, b2_0, g2_1, b2_1,
                       g2_2, b2_2)
    src_nbr_desc = nbr_desc[0]
    dst_nbr_desc = nbr_desc[1]

    sdnk, dsnk = _nbr_cos(src_nbr_desc, dst_nbr_desc, idx)

    bc = lambda v, ch: jnp.broadcast_to(v[:, None], (B, K, N, ch))
    tr = lambda v: jnp.transpose(v, (0, 2, 1))[..., None]
    feats2 = jnp.concatenate([
        krel, kdist, bc(src_xyz, 3), kxyz,
        bc(src_desc_t, C), kdesc, bc(src_weights[..., None], 1), kw,
        tr(sdk), tr(ddk), tr(sdnk), tr(dsnk),
    ], axis=-1)
    c2 = 2 * C
    x0 = feats2.reshape(P2D, c2 + 16)
    y2, st2 = _big3(x0, W1_0, W1_1, W1_2, g1_0, b1_0, g1_1, b1_1)
    corres_xyz, wout = _attn1_head(y2, st2, g1_2, b1_2, kxyz,
                                   Wm1, bm1, gm1, bnm1, Wm2, bm2, gm2,
                                   bnm2, Wm3, bm3)
    weights = jnp.transpose(wout, (1, 0))
    return corres_xyz, weights


# speedup vs baseline: 5.5428x; 1.1006x over previous
import functools

import jax
import jax.numpy as jnp
from jax.experimental import pallas as pl
from jax.experimental.pallas import tpu as pltpu

B, N, C, K = 2, 512, 128, 16
EPS = 1e-5
P2D = B * K * N
P1D = B * N


def _dot(a, b, dims):
    return jax.lax.dot_general(a, b, (dims, ((), ())),
                               preferred_element_type=jnp.float32)


def _dot_hi(a, b, dims):
    return jax.lax.dot_general(a, b, (dims, ((), ())),
                               precision=jax.lax.Precision.HIGHEST,
                               preferred_element_type=jnp.float32)


def _topk_min(d, k):
    n = d.shape[1]
    col_i = jax.lax.broadcasted_iota(jnp.int32, d.shape, 1)
    idxs = []
    for _ in range(k):
        m = jnp.min(d, axis=1, keepdims=True)
        cand = jnp.where(d == m, col_i, n)
        ij = jnp.min(cand, axis=1, keepdims=True)
        idxs.append(ij)
        d = jnp.where(col_i == ij, jnp.float32(jnp.inf), d)
    return idxs


def _desc_knn_body(src_ref, dst_ref, dxyz_ref, dw_ref, sxyz_ref,
                   idx_ref, sdk_ref, ddk_ref, kdesc_ref, small_ref):
    col_i = jax.lax.broadcasted_iota(jnp.int32, (N, N), 1)
    dst = dst_ref[0]
    src = src_ref[0]
    S = _dot(src, dst, ((1,), (1,)))
    ns2 = jnp.sum(src * src, axis=1, keepdims=True)
    nd2_row = _dot_hi(jnp.ones((1, C), jnp.float32), dst * dst,
                      ((1,), (1,)))
    d = ns2 + nd2_row - 2.0 * S
    idxs = _topk_min(d, K)
    idx_ref[0] = jnp.concatenate(idxs, axis=1)

    ns = jnp.sqrt(ns2)
    nd_row = jnp.sqrt(nd2_row)
    cm = S / (ns * nd_row + 1e-6)
    rowmax = jnp.max(cm, axis=1, keepdims=True)
    colmax = jnp.max(cm, axis=0, keepdims=True)
    sxyz = sxyz_ref[0]
    dxyz = dxyz_ref[0]
    dwv = dw_ref[0]
    gjs, cgs = [], []
    for j in range(K):
        oh = (col_i == idxs[j]).astype(jnp.float32)
        gjs.append(jnp.sum(cm * oh, axis=1, keepdims=True))
        cgs.append(jnp.sum(colmax * oh, axis=1, keepdims=True))
        kdesc_ref[0, j] = _dot(oh, dst, ((1,), (0,)))
        gxyz = _dot(oh, dxyz, ((1,), (0,)))
        rel = gxyz - sxyz
        dist = jnp.sqrt(jnp.sum(rel * rel, axis=1, keepdims=True))
        gw = _dot(oh, dwv, ((1,), (0,)))
        small_ref[0, j] = jnp.concatenate([rel, dist, gxyz, gw], axis=1)
    gj = jnp.concatenate(gjs, axis=1)
    cg = jnp.concatenate(cgs, axis=1)
    sdk_ref[0] = gj / (rowmax + 1e-6)
    ddk_ref[0] = gj / (cg + 1e-6)


def _desc_knn(src_desc_t, dst_desc_t, dst_xyz, dst_w, src_xyz):
    f32 = jnp.float32
    outs = (
        jax.ShapeDtypeStruct((B, N, K), jnp.int32),
        jax.ShapeDtypeStruct((B, N, K), f32),
        jax.ShapeDtypeStruct((B, N, K), f32),
        jax.ShapeDtypeStruct((B, K, N, C), f32),
        jax.ShapeDtypeStruct((B, K, N, 8), f32),
    )
    i3 = lambda b: (b, 0, 0)
    i4 = lambda b: (b, 0, 0, 0)
    return pl.pallas_call(
        _desc_knn_body,
        grid=(B,),
        in_specs=[
            pl.BlockSpec((1, N, C), i3),
            pl.BlockSpec((1, N, C), i3),
            pl.BlockSpec((1, N, 3), i3),
            pl.BlockSpec((1, N, 1), i3),
            pl.BlockSpec((1, N, 3), i3),
        ],
        out_specs=[
            pl.BlockSpec((1, N, K), i3),
            pl.BlockSpec((1, N, K), i3),
            pl.BlockSpec((1, N, K), i3),
            pl.BlockSpec((1, K, N, C), i4),
            pl.BlockSpec((1, K, N, 8), i4),
        ],
        out_shape=outs,
    )(src_desc_t, dst_desc_t, dst_xyz, dst_w, src_xyz)


def _xyz_knn_body(xyz_ref, desc_ref, gdesc_ref, small_ref):
    xyz = xyz_ref[0]
    desc = desc_ref[0]
    col_i = jax.lax.broadcasted_iota(jnp.int32, (N, N), 1)
    Pm = _dot(xyz, xyz, ((1,), (1,)))
    ss = jnp.sum(xyz * xyz, axis=1, keepdims=True)
    ss_row = _dot_hi(jnp.ones((1, 3), jnp.float32), xyz * xyz,
                     ((1,), (1,)))
    d = ss + ss_row - 2.0 * Pm
    idxs = _topk_min(d, K)
    for j in range(K):
        oh = (col_i == idxs[j]).astype(jnp.float32)
        gdesc_ref[0, j] = _dot(oh, desc, ((1,), (0,)))
        gxyz = _dot(oh, xyz, ((1,), (0,)))
        rel = gxyz - xyz
        dist = jnp.sqrt(jnp.sum(rel * rel, axis=1, keepdims=True))
        small_ref[0, j] = jnp.concatenate([rel, dist], axis=1)


def _xyz_knn(xyz, desc_t):
    g = xyz.shape[0]
    f32 = jnp.float32
    outs = (
        jax.ShapeDtypeStruct((g, K, N, C), f32),
        jax.ShapeDtypeStruct((g, K, N, 4), f32),
    )
    i3 = lambda b: (b, 0, 0)
    i4 = lambda b: (b, 0, 0, 0)
    return pl.pallas_call(
        _xyz_knn_body,
        grid=(g,),
        in_specs=[pl.BlockSpec((1, N, 3), i3), pl.BlockSpec((1, N, C), i3)],
        out_specs=[
            pl.BlockSpec((1, K, N, C), i4),
            pl.BlockSpec((1, K, N, 4), i4),
        ],
        out_shape=outs,
    )(xyz, desc_t)


def _nbr_cos_body(s_ref, d_ref, idx_ref, sdnk_ref, dsnk_ref):
    s = s_ref[0]
    t = d_ref[0]
    S2 = _dot(s, t, ((1,), (1,)))
    ns = jnp.sqrt(jnp.sum(s * s, axis=1, keepdims=True))
    nd_row = jnp.sqrt(_dot_hi(jnp.ones((1, C), jnp.float32), t * t,
                              ((1,), (1,))))
    cm = S2 / (ns * nd_row + 1e-6)
    rowmax = jnp.max(cm, axis=1, keepdims=True)
    colmax = jnp.max(cm, axis=0, keepdims=True)
    idx = idx_ref[0]
    col_i = jax.lax.broadcasted_iota(jnp.int32, (N, N), 1)
    gjs, cgs = [], []
    for j in range(K):
        ij = idx[:, j:j + 1]
        oh = (col_i == ij).astype(jnp.float32)
        gjs.append(jnp.sum(cm * oh, axis=1, keepdims=True))
        cgs.append(jnp.sum(colmax * oh, axis=1, keepdims=True))
    gj = jnp.concatenate(gjs, axis=1)
    cg = jnp.concatenate(cgs, axis=1)
    sdnk_ref[0] = gj / (rowmax + 1e-6)
    dsnk_ref[0] = gj / (cg + 1e-6)


def _nbr_cos(src_nbr_desc, dst_nbr_desc, idx):
    f32 = jnp.float32
    i3 = lambda b: (b, 0, 0)
    i4 = lambda b: (b, 0, 0, 0)
    return pl.pallas_call(
        _nbr_cos_body,
        grid=(B,),
        in_specs=[
            pl.BlockSpec((1, N, C), i3),
            pl.BlockSpec((1, N, C), i3),
            pl.BlockSpec((1, N, K), i3),
        ],
        out_specs=[
            pl.BlockSpec((1, N, K), i3),
            pl.BlockSpec((1, N, K), i3),
        ],
        out_shape=(jax.ShapeDtypeStruct((B, N, K), f32),
                   jax.ShapeDtypeStruct((B, N, K), f32)),
    )(src_nbr_desc, dst_nbr_desc, idx)


def _bn_apply(y, g, b, n_pos):
    s0 = jnp.sum(y, axis=0, keepdims=True)
    s1 = jnp.sum(y * y, axis=0, keepdims=True)
    mean = s0 / n_pos
    var = s1 / n_pos - mean * mean
    inv = 1.0 / jnp.sqrt(var + EPS)
    return jnp.maximum((y - mean) * inv * g + b, 0.0)


def _convs2_body(x_ref, w0_ref, w1_ref, w2_ref, g0_ref, b0_ref, g1_ref,
                 b1_ref, g2_ref, b2_ref, out_ref):
    n_pos = float(P2D)
    x = x_ref[0]
    y = _dot(x, w0_ref[...], ((1,), (1,)))
    y = _bn_apply(y, g0_ref[...], b0_ref[...], n_pos)
    y = _dot(y, w1_ref[...], ((1,), (1,)))
    y = _bn_apply(y, g1_ref[...], b1_ref[...], n_pos)
    y = _dot(y, w2_ref[...], ((1,), (1,)))
    y = _bn_apply(y, g2_ref[...], b2_ref[...], n_pos)
    for b in range(B):
        cms = []
        for j in range(K):
            r0 = (b * K + j) * N
            cms.append(jnp.max(y[r0:r0 + N, :], axis=1, keepdims=True))
        a = jnp.concatenate(cms, axis=1)
        m = jnp.max(a, axis=1, keepdims=True)
        e = jnp.exp(a - m)
        aw = e / jnp.sum(e, axis=1, keepdims=True)
        acc = jnp.zeros((N, C), jnp.float32)
        for j in range(K):
            r0 = (b * K + j) * N
            acc = acc + aw[:, j:j + 1] * x[r0:r0 + N, 0:C]
        out_ref[0, b] = acc


def _convs2(feats, w0, w1, w2, g0, b0, g1, b1, g2, b2):
    f32 = jnp.float32
    r = lambda v: v.reshape(1, -1)
    cw = lambda shp: pl.BlockSpec(shp, lambda s: (0, 0))
    return pl.pallas_call(
        _convs2_body,
        grid=(2,),
        in_specs=[pl.BlockSpec((1, P2D, C + 4), lambda s: (s, 0, 0)),
                  cw(w0.shape), cw(w1.shape), cw(w2.shape),
                  cw((1, C)), cw((1, C)), cw((1, C)), cw((1, C)),
                  cw((1, C)), cw((1, C))],
        out_specs=pl.BlockSpec((1, B, N, C), lambda s: (s, 0, 0, 0)),
        out_shape=jax.ShapeDtypeStruct((2, B, N, C), f32),
    )(feats, w0, w1, w2, r(g0), r(b0), r(g1), r(b1), r(g2), r(b2))


_TILE = 2048
_NT = P2D // _TILE


def _big3_body(x_ref, w0_ref, w1_ref, w2_ref, g0_ref, b0_ref, g1_ref,
               b1_ref, y2_ref, st2_ref, ping, pong, st0, st1):
    p = pl.program_id(0)
    t = pl.program_id(1)
    n_pos = float(P2D)

    @pl.when((p == 0) & (t == 0))
    def _init():
        st0[...] = jnp.zeros_like(st0)
        st1[...] = jnp.zeros_like(st1)

    row = t * _TILE

    @pl.when(p == 0)
    def _l0():
        y = _dot(x_ref[...], w0_ref[...], ((1,), (1,)))
        ping[pl.ds(row, _TILE), :] = y
        st0[...] = st0[...] + jnp.concatenate(
            [jnp.sum(y, axis=0, keepdims=True),
             jnp.sum(y * y, axis=0, keepdims=True)], axis=0)

    @pl.when(p == 1)
    def _l1():
        st = st0[...]
        mean = st[0:1, :] / n_pos
        var = st[1:2, :] / n_pos - mean * mean
        inv = 1.0 / jnp.sqrt(var + EPS)
        xn = jnp.maximum((ping[pl.ds(row, _TILE), :] - mean) * inv
                         * g0_ref[...] + b0_ref[...], 0.0)
        y = _dot(xn, w1_ref[...], ((1,), (1,)))
        pong[pl.ds(row, _TILE), :] = y
        st1[...] = st1[...] + jnp.concatenate(
            [jnp.sum(y, axis=0, keepdims=True),
             jnp.sum(y * y, axis=0, keepdims=True)], axis=0)

    @pl.when(p == 2)
    def _l2():
        st = st1[...]
        mean = st[0:1, :] / n_pos
        var = st[1:2, :] / n_pos - mean * mean
        inv = 1.0 / jnp.sqrt(var + EPS)
        xn = jnp.maximum((pong[pl.ds(row, _TILE), :] - mean) * inv
                         * g1_ref[...] + b1_ref[...], 0.0)
        y = _dot(xn, w2_ref[...], ((1,), (1,)))
        y2_ref[...] = y
        st2_ref[0] = jnp.concatenate(
            [jnp.sum(y, axis=0, keepdims=True),
             jnp.sum(y * y, axis=0, keepdims=True)], axis=0)


def _big3(x, w0, w1, w2, g0, b0, g1, b1):
    cin = x.shape[1]
    c2 = w2.shape[0]
    f32 = jnp.float32
    r = lambda v: v.reshape(1, -1)
    cw = lambda shp: pl.BlockSpec(shp, lambda p, t: (0, 0))
    sel = lambda pp, tt: (jnp.where(pp == 0, tt, 0), 0)
    sely = lambda pp, tt: (jnp.where(pp == 2, tt, 0), 0)
    selst = lambda pp, tt: (jnp.where(pp == 2, tt, 0), 0, 0)
    return pl.pallas_call(
        _big3_body,
        grid=(3, _NT),
        in_specs=[pl.BlockSpec((_TILE, cin), sel),
                  cw(w0.shape), cw(w1.shape), cw(w2.shape),
                  cw((1, c2)), cw((1, c2)), cw((1, c2)), cw((1, c2))],
        out_specs=[pl.BlockSpec((_TILE, c2), sely),
                   pl.BlockSpec((1, 2, c2), selst)],
        out_shape=(jax.ShapeDtypeStruct((P2D, c2), f32),
                   jax.ShapeDtypeStruct((_NT, 2, c2), f32)),
        scratch_shapes=[pltpu.VMEM((P2D, c2), f32),
                        pltpu.VMEM((P2D, c2), f32),
                        pltpu.VMEM((2, c2), f32),
                        pltpu.VMEM((2, c2), f32)],
    )(x, w0, w1, w2, r(g0), r(b0), r(g1), r(b1))


def _attn1_head_body(y_ref, st_ref, g_ref, b_ref, kxyz_ref,
                     w1_ref, b1_ref, g1_ref, n1_ref,
                     w2_ref, b2_ref, g2_ref, n2_ref, w3_ref, b3_ref,
                     corres_ref, wout_ref):
    c2 = 2 * C
    n_pos = float(P2D)
    st = jnp.sum(st_ref[...], axis=0)
    mean = st[0:1, :] / n_pos
    var = st[1:2, :] / n_pos - mean * mean
    inv = 1.0 / jnp.sqrt(var + EPS)
    g = g_ref[...]
    bb = b_ref[...]
    y = y_ref[...]

    def xn_at(b, j):
        r0 = (b * K + j) * N
        return jnp.maximum((y[r0:r0 + N, :] - mean) * inv * g + bb, 0.0)

    att_rows = []
    for b in range(B):
        cms = []
        for j in range(K):
            cms.append(jnp.max(xn_at(b, j), axis=1, keepdims=True))
        a = jnp.concatenate(cms, axis=1)
        m = jnp.max(a, axis=1, keepdims=True)
        e = jnp.exp(a - m)
        aw = e / jnp.sum(e, axis=1, keepdims=True)
        corres = jnp.zeros((N, 3), jnp.float32)
        att = jnp.zeros((N, c2), jnp.float32)
        for j in range(K):
            awj = aw[:, j:j + 1]
            corres = corres + awj * kxyz_ref[0, b * K + j]
            att = att + awj * xn_at(b, j)
        corres_ref[b] = corres
        att_rows.append(att)
    x = jnp.concatenate(att_rows, axis=0)

    def cbr(xx, w_ref, bias_ref, gg_ref, bt_ref):
        yy = _dot(xx, w_ref[...], ((1,), (1,))) + bias_ref[...]
        mm = jnp.mean(yy, axis=0, keepdims=True)
        vv = jnp.mean((yy - mm) * (yy - mm), axis=0, keepdims=True)
        return jnp.maximum((yy - mm) / jnp.sqrt(vv + EPS) * gg_ref[...]
                           + bt_ref[...], 0.0)

    x = cbr(x, w1_ref, b1_ref, g1_ref, n1_ref)
    x = cbr(x, w2_ref, b2_ref, g2_ref, n2_ref)
    w = jnp.sum(x * w3_ref[...], axis=1, keepdims=True) + b3_ref[0, 0]
    s = 1.0 / (1.0 + jnp.exp(-w))
    wout_ref[...] = jnp.concatenate(
        [s[b * N:(b + 1) * N, :] for b in range(B)], axis=1)


def _attn1_head(y2, st2, g, b, kxyz, wm1, bm1, gm1, bnm1, wm2, bm2, gm2,
                bnm2, wm3, bm3):
    c2 = 2 * C
    f32 = jnp.float32
    r = lambda v: v.reshape(1, -1)
    kx = kxyz.reshape(1, B * K, N, 3)
    return pl.pallas_call(
        _attn1_head_body,
        out_shape=(jax.ShapeDtypeStruct((B, N, 3), f32),
                   jax.ShapeDtypeStruct((N, B), f32)),
    )(y2, st2, r(g), r(b), kx, wm1, r(bm1), r(gm1), r(bnm1),
      wm2, r(bm2), r(gm2), r(bnm2), wm3, r(bm3))


def kernel(src_xyz, src_desc, dst_xyz, dst_desc, src_weights, dst_weights,
           W2_0, W2_1, W2_2, g2_0, b2_0, g2_1, b2_1, g2_2, b2_2,
           W1_0, W1_1, W1_2, g1_0, b1_0, g1_1, b1_1, g1_2, b1_2,
           Wm1, bm1, gm1, bnm1, Wm2, bm2, gm2, bnm2, Wm3, bm3):
    src_desc_t = jnp.transpose(src_desc, (0, 2, 1))
    dst_desc_t = jnp.transpose(dst_desc, (0, 2, 1))

    (idx, sdk, ddk, kdesc, ksmall) = _desc_knn(
        src_desc_t, dst_desc_t, dst_xyz, dst_weights[..., None], src_xyz)
    krel = ksmall[..., 0:3]
    kdist = ksmall[..., 3:4]
    kxyz = ksmall[..., 4:7]
    kw = ksmall[..., 7:8]

    xyz_st = jnp.concatenate([src_xyz, dst_xyz], axis=0)
    desc_st = jnp.concatenate([src_desc_t, dst_desc_t], axis=0)
    gdesc, gsmall = _xyz_knn(xyz_st, desc_st)

    feats1 = jnp.concatenate([gdesc, gsmall], axis=-1)
    nbr_desc = _convs2(feats1.reshape(2, P2D, C + 4),
                       W2_0, W2_1, W2_2, g2_0, b2_0, g2_1, b2_1,
                       g2_2, b2_2)
    src_nbr_desc = nbr_desc[0]
    dst_nbr_desc = nbr_desc[1]

    sdnk, dsnk = _nbr_cos(src_nbr_desc, dst_nbr_desc, idx)

    bc = lambda v, ch: jnp.broadcast_to(v[:, None], (B, K, N, ch))
    tr = lambda v: jnp.transpose(v, (0, 2, 1))[..., None]
    feats2 = jnp.concatenate([
        krel, kdist, bc(src_xyz, 3), kxyz,
        bc(src_desc_t, C), kdesc, bc(src_weights[..., None], 1), kw,
        tr(sdk), tr(ddk), tr(sdnk), tr(dsnk),
    ], axis=-1)
    c2 = 2 * C
    x0 = feats2.reshape(P2D, c2 + 16)
    y2, st2 = _big3(x0, W1_0, W1_1, W1_2, g1_0, b1_0, g1_1, b1_1)
    corres_xyz, wout = _attn1_head(y2, st2, g1_2, b1_2, kxyz,
                                   Wm1, bm1, gm1, bnm1, Wm2, bm2, gm2,
                                   bnm2, Wm3, bm3)
    weights = jnp.transpose(wout, (1, 0))
    return corres_xyz, weights
